# baseline (speedup 1.0000x reference)
import functools

import jax
import jax.numpy as jnp
from jax import lax
from jax.experimental import pallas as pl
from jax.experimental.pallas import tpu as pltpu
from jax.experimental.pallas import tpu_sc as plsc

H = 768
NV = H // 16
EPS = 1e-12
C = 32


def _make_sc_kernel(BT, S, V, T):
    info = plsc.get_sparse_core_info()
    NC, NS = info.num_cores, info.num_subcores
    NW = NC * NS
    TPW = BT // NW
    NCHUNK = TPW // C

    mesh = plsc.VectorSubcoreMesh(core_axis_name="c", subcore_axis_name="s")

    @functools.partial(
        pl.kernel,
        mesh=mesh,
        out_type=jax.ShapeDtypeStruct((BT, H), jnp.float32),
        scratch_types=[
            pltpu.VMEM((C,), jnp.int32),
            pltpu.VMEM((C,), jnp.int32),
            pltpu.VMEM((C, H), jnp.float32),
            pltpu.VMEM((C, H), jnp.float32),
            pltpu.VMEM((C, H), jnp.float32),
            pltpu.VMEM((H,), jnp.float32),
            pltpu.VMEM((H,), jnp.float32),
            pltpu.SemaphoreType.DMA,
        ],
    )
    def k(ids_hbm, tt_hbm, word_hbm, pos_hbm, type_hbm, gamma_hbm, beta_hbm,
          out_hbm, idx_v, tt_v, word_v, pos_v, type_v, gamma_v, beta_v, sem):
        wid = lax.axis_index("s") * NC + lax.axis_index("c")
        base = wid * TPW
        pltpu.sync_copy(gamma_hbm, gamma_v)
        pltpu.sync_copy(beta_hbm, beta_v)

        def chunk_body(c, carry):
            tok0 = base + c * C
            p0 = lax.rem(tok0, S)
            pltpu.sync_copy(ids_hbm.at[pl.ds(tok0, C)], idx_v)
            pltpu.sync_copy(tt_hbm.at[pl.ds(tok0, C)], tt_v)
            cw = pltpu.async_copy(word_hbm.at[idx_v], word_v, sem)
            ct = pltpu.async_copy(type_hbm.at[tt_v], type_v, sem)
            cp = pltpu.async_copy(pos_hbm.at[pl.ds(p0, C), :], pos_v, sem)
            cw.wait()
            ct.wait()
            cp.wait()

            def token_body(i, tcarry):
                a0 = jnp.zeros((16,), jnp.float32)
                a1 = jnp.zeros((16,), jnp.float32)
                q0 = jnp.zeros((16,), jnp.float32)
                q1 = jnp.zeros((16,), jnp.float32)
                for j in range(NV):
                    w = word_v[i, pl.ds(j * 16, 16)]
                    p = pos_v[i, pl.ds(j * 16, 16)]
                    t = type_v[i, pl.ds(j * 16, 16)]
                    s = (w + p) + t
                    word_v[i, pl.ds(j * 16, 16)] = s
                    if j % 2 == 0:
                        a0 = a0 + s
                        q0 = q0 + s * s
                    else:
                        a1 = a1 + s
                        q1 = q1 + s * s
                lanes = lax.iota(jnp.int32, 16)
                sumv = a0 + a1
                sqv = q0 + q1
                for k in (1, 2, 4, 8):
                    perm = lanes ^ k
                    sumv = sumv + sumv.at[perm].get(mode="promise_in_bounds")
                    sqv = sqv + sqv.at[perm].get(mode="promise_in_bounds")
                mvec = sumv * (1.0 / H)
                v16 = sqv * (1.0 / H) - mvec * mvec + EPS
                iv = lax.bitcast_convert_type(v16, jnp.int32)
                y = lax.bitcast_convert_type(
                    jnp.int32(0x5F3759DF) - (iv >> 1), jnp.float32)
                for _ in range(4):
                    y = y * (1.5 - 0.5 * v16 * y * y)
                for j in range(NV):
                    s = word_v[i, pl.ds(j * 16, 16)]
                    g = gamma_v[pl.ds(j * 16, 16)]
                    b = beta_v[pl.ds(j * 16, 16)]
                    word_v[i, pl.ds(j * 16, 16)] = (s - mvec) * y * g + b
                return tcarry

            lax.fori_loop(0, C, token_body, 0)
            pltpu.sync_copy(word_v, out_hbm.at[pl.ds(tok0, C), :])
            return carry

        lax.fori_loop(0, NCHUNK, chunk_body, 0)

    return k


def kernel(input_ids, token_type_ids, word_emb, pos_emb, type_emb, gamma, beta):
    B, S = input_ids.shape
    V = word_emb.shape[0]
    T = type_emb.shape[0]
    BT = B * S
    ids = input_ids.reshape(BT).astype(jnp.int32)
    tts = token_type_ids.reshape(BT).astype(jnp.int32)
    k = _make_sc_kernel(BT, S, V, T)
    out = k(ids, tts, word_emb, pos_emb, type_emb, gamma, beta)
    return out.reshape(B, S, H)


# speedup vs baseline: 3.7106x; 3.7106x over previous
import functools

import jax
import jax.numpy as jnp
from jax import lax
from jax.experimental import pallas as pl
from jax.experimental.pallas import tpu as pltpu
from jax.experimental.pallas import tpu_sc as plsc

H = 768
EPS = 1e-12
C = 64


def _make_sc_gather(BT, V):
    info = plsc.get_sparse_core_info()
    NC, NS = info.num_cores, info.num_subcores
    NW = NC * NS
    TPW = BT // NW
    NCHUNK = TPW // C

    mesh = plsc.VectorSubcoreMesh(core_axis_name="c", subcore_axis_name="s")

    @functools.partial(
        pl.kernel,
        mesh=mesh,
        out_type=jax.ShapeDtypeStruct((BT, H), jnp.float32),
        scratch_types=[
            pltpu.VMEM((C,), jnp.int32),
            pltpu.VMEM((C,), jnp.int32),
            pltpu.VMEM((C, H), jnp.float32),
            pltpu.VMEM((C, H), jnp.float32),
            pltpu.SemaphoreType.DMA,
            pltpu.SemaphoreType.DMA,
            pltpu.SemaphoreType.DMA,
            pltpu.SemaphoreType.DMA,
        ],
    )
    def k(ids_hbm, word_hbm, out_hbm,
          idx0, idx1, buf0, buf1, gs0, gs1, ss0, ss1):
        wid = lax.axis_index("s") * NC + lax.axis_index("c")
        base = wid * TPW
        idx = (idx0, idx1)
        buf = (buf0, buf1)
        gsem = (gs0, gs1)
        ssem = (ss0, ss1)

        gathers = [None, None]
        scatters = [None, None]
        for c in range(NCHUNK):
            r = c & 1
            if scatters[r] is not None:
                scatters[r].wait()
            tok0 = base + c * C
            pltpu.sync_copy(ids_hbm.at[pl.ds(tok0, C)], idx[r])
            gathers[r] = pltpu.async_copy(word_hbm.at[idx[r]], buf[r], gsem[r])
            o = 1 - r
            if gathers[o] is not None:
                gathers[o].wait()
                otok0 = base + (c - 1) * C
                scatters[o] = pltpu.async_copy(
                    buf[o], out_hbm.at[pl.ds(otok0, C), :], ssem[o])
                gathers[o] = None
        r = (NCHUNK - 1) & 1
        gathers[r].wait()
        tok0 = base + (NCHUNK - 1) * C
        scatters[r] = pltpu.async_copy(
            buf[r], out_hbm.at[pl.ds(tok0, C), :], ssem[r])
        scatters[0].wait()
        scatters[1].wait()

    return k


def _make_tc_ln(BT, S, B, T):
    TOK = 256
    SCH = S // TOK
    grid = (SCH, B)

    def body(g_ref, pos_ref, tt_ref, type_ref, gam_ref, bet_ref, o_ref):
        x = g_ref[...] + pos_ref[...]
        tt = tt_ref[...]
        t0 = type_ref[0, :][None, :]
        t1 = type_ref[1, :][None, :]
        x = x + jnp.where(tt == 0.0, t0, t1)
        mean = jnp.mean(x, axis=-1, keepdims=True)
        c = x - mean
        var = jnp.mean(c * c, axis=-1, keepdims=True)
        inv = lax.rsqrt(var + EPS)
        o_ref[...] = gam_ref[...] * (c * inv) + bet_ref[...]

    return pl.pallas_call(
        body,
        grid=grid,
        in_specs=[
            pl.BlockSpec((TOK, H), lambda sc, b: (b * SCH + sc, 0)),
            pl.BlockSpec((TOK, H), lambda sc, b: (sc, 0)),
            pl.BlockSpec((TOK, 1), lambda sc, b: (b * SCH + sc, 0)),
            pl.BlockSpec((T, H), lambda sc, b: (0, 0)),
            pl.BlockSpec((1, H), lambda sc, b: (0, 0)),
            pl.BlockSpec((1, H), lambda sc, b: (0, 0)),
        ],
        out_specs=pl.BlockSpec((TOK, H), lambda sc, b: (b * SCH + sc, 0)),
        out_shape=jax.ShapeDtypeStruct((BT, H), jnp.float32),
    )


def kernel(input_ids, token_type_ids, word_emb, pos_emb, type_emb, gamma, beta):
    B, S = input_ids.shape
    V = word_emb.shape[0]
    T = type_emb.shape[0]
    BT = B * S
    ids = input_ids.reshape(BT).astype(jnp.int32)
    tt_f = token_type_ids.reshape(BT, 1).astype(jnp.float32)
    gathered = _make_sc_gather(BT, V)(ids, word_emb)
    out = _make_tc_ln(BT, S, B, T)(
        gathered, pos_emb, tt_f, type_emb,
        gamma.reshape(1, H), beta.reshape(1, H))
    return out.reshape(B, S, H)
